# Initial kernel scaffold; baseline (speedup 1.0000x reference)
import jax
import jax.numpy as jnp
from jax.experimental import pallas as pl


def kernel(domain_indices, counts, embd_weight, weights):
    raise NotImplementedError("write your pallas kernel here")


# speedup vs baseline: 79.1606x; 79.1606x over previous
import functools

import jax
import jax.numpy as jnp
from jax import lax
from jax.experimental import pallas as pl
from jax.experimental.pallas import tpu as pltpu
from jax.experimental.pallas import tpu_sc as plsc

B = 16384
H = 200
V = 1002
VPAD = 1008
E = 4
L = 16
NC = 2
NS = 16
NW = NC * NS
ROWS_PER_W = B // NW
CHUNK = 16
NCHUNK = ROWS_PER_W // CHUNK
CELEM = CHUNK * H

_mesh = plsc.VectorSubcoreMesh(core_axis_name="c", subcore_axis_name="s")


@functools.partial(
    pl.kernel,
    mesh=_mesh,
    out_type=jax.ShapeDtypeStruct((B,), jnp.float32),
    compiler_params=pltpu.CompilerParams(needs_layout_passes=False),
    scratch_types=[
        pltpu.VMEM((E * VPAD,), jnp.float32),
        pltpu.VMEM((E * L,), jnp.float32),
        pltpu.VMEM((VPAD,), jnp.float32),
        pltpu.VMEM((CELEM,), jnp.int32),
        pltpu.VMEM((CELEM,), jnp.float32),
        pltpu.VMEM((L,), jnp.float32),
    ],
)
def _dwc_kernel(idx_hbm, cnt_hbm, tab_hbm, w_hbm, out_hbm,
                tab_v, w_v, comb_v, idx_v, cnt_v, out_v):
    cid = lax.axis_index("c")
    sid = lax.axis_index("s")
    wid = sid * NC + cid
    lanes = lax.iota(jnp.int32, L)

    pltpu.sync_copy(tab_hbm, tab_v)
    pltpu.sync_copy(w_hbm, w_v)

    wsplat = [w_v[pl.ds(e * L, L)] for e in range(E)]

    def fold_body(k, carry):
        base = k * L
        acc = jnp.zeros((L,), jnp.float32)
        for e in range(E):
            acc = acc + tab_v[pl.ds(e * VPAD + base, L)] * wsplat[e]
        comb_v[pl.ds(base, L)] = acc
        return carry

    lax.fori_loop(0, VPAD // L, fold_body, 0)

    elem0 = wid * ROWS_PER_W * H
    lane_off = lanes * H

    def chunk_body(c, carry):
        e0 = elem0 + c * CELEM
        pltpu.sync_copy(idx_hbm.at[pl.ds(e0, CELEM)], idx_v)
        pltpu.sync_copy(cnt_hbm.at[pl.ds(e0, CELEM)], cnt_v)

        def d_body(d, acc):
            pos = lane_off + d
            ii = plsc.load_gather(idx_v, [pos])
            cc = plsc.load_gather(cnt_v, [pos])
            vv = plsc.load_gather(comb_v, [ii])
            return acc + cc * vv

        acc = lax.fori_loop(0, H, d_body, jnp.zeros((L,), jnp.float32))
        out_v[...] = acc
        r0 = wid * ROWS_PER_W + c * CHUNK
        pltpu.sync_copy(out_v, out_hbm.at[pl.ds(r0, CHUNK)])
        return carry

    lax.fori_loop(0, NCHUNK, chunk_body, 0)


def kernel(domain_indices, counts, embd_weight, weights):
    idx_flat = domain_indices.reshape(B * H)
    cnt_flat = counts.reshape(B * H)
    tab_t = jnp.zeros((E, VPAD), jnp.float32).at[:, :V].set(embd_weight.T)
    tab_flat = tab_t.reshape(E * VPAD)
    w_bcast = jnp.broadcast_to(weights.reshape(E, 1), (E, L)).reshape(E * L)
    out = _dwc_kernel(idx_flat, cnt_flat, tab_flat, w_bcast)
    return out.reshape(B, 1)


# speedup vs baseline: 128.6471x; 1.6251x over previous
import functools

import jax
import jax.numpy as jnp
from jax import lax
from jax.experimental import pallas as pl
from jax.experimental.pallas import tpu as pltpu
from jax.experimental.pallas import tpu_sc as plsc

B = 16384
H = 200
V = 1002
VPAD = 1008
E = 4
L = 16
NC = 2
NS = 16
NW = NC * NS
ROWS_PER_W = B // NW
CHUNK = 16
NCHUNK = ROWS_PER_W // CHUNK
CELEM = CHUNK * H

_mesh = plsc.VectorSubcoreMesh(core_axis_name="c", subcore_axis_name="s")


@functools.partial(
    pl.kernel,
    mesh=_mesh,
    out_type=jax.ShapeDtypeStruct((B,), jnp.float32),
    compiler_params=pltpu.CompilerParams(needs_layout_passes=False),
    scratch_types=[
        pltpu.VMEM((E * VPAD,), jnp.float32),
        pltpu.VMEM((E * L,), jnp.float32),
        pltpu.VMEM((VPAD,), jnp.float32),
        pltpu.VMEM((CELEM,), jnp.int32),
        pltpu.VMEM((CELEM,), jnp.int32),
        pltpu.VMEM((CELEM,), jnp.float32),
        pltpu.VMEM((CELEM,), jnp.float32),
        pltpu.VMEM((ROWS_PER_W,), jnp.float32),
        pltpu.SemaphoreType.DMA,
        pltpu.SemaphoreType.DMA,
    ],
)
def _dwc_kernel(idx_hbm, cnt_hbm, tab_hbm, w_hbm, out_hbm,
                tab_v, w_v, comb_v, idx0_v, idx1_v, cnt0_v, cnt1_v,
                res_v, sem0, sem1):
    cid = lax.axis_index("c")
    sid = lax.axis_index("s")
    wid = sid * NC + cid
    lanes = lax.iota(jnp.int32, L)

    pltpu.sync_copy(tab_hbm, tab_v)
    pltpu.sync_copy(w_hbm, w_v)

    wsplat = [w_v[pl.ds(e * L, L)] for e in range(E)]

    def fold_body(k, carry):
        base = k * L
        acc = jnp.zeros((L,), jnp.float32)
        for e in range(E):
            acc = acc + tab_v[pl.ds(e * VPAD + base, L)] * wsplat[e]
        comb_v[pl.ds(base, L)] = acc
        return carry

    lax.fori_loop(0, VPAD // L, fold_body, 0)

    elem0 = wid * ROWS_PER_W * H
    lane_off = lanes * H
    UNROLL = 8

    def start_fetch(c, ibuf, cbuf, sem):
        e0 = elem0 + jnp.minimum(c, NCHUNK - 1) * CELEM
        pltpu.async_copy(idx_hbm.at[pl.ds(e0, CELEM)], ibuf, sem)
        pltpu.async_copy(cnt_hbm.at[pl.ds(e0, CELEM)], cbuf, sem)

    def wait_fetch(ibuf, cbuf, sem):
        pltpu.make_async_copy(idx_hbm.at[pl.ds(0, CELEM)], ibuf, sem).wait()
        pltpu.make_async_copy(cnt_hbm.at[pl.ds(0, CELEM)], cbuf, sem).wait()

    def compute(c, ibuf, cbuf):
        def d_body(dd, acc):
            pos0 = lane_off + dd * UNROLL
            for j in range(UNROLL):
                pos = pos0 + j
                ii = plsc.load_gather(ibuf, [pos])
                cc = plsc.load_gather(cbuf, [pos])
                vv = plsc.load_gather(comb_v, [ii])
                acc = acc + cc * vv
            return acc

        acc = lax.fori_loop(0, H // UNROLL, d_body,
                            jnp.zeros((L,), jnp.float32))
        res_v[pl.ds(c * CHUNK, CHUNK)] = acc

    start_fetch(0, idx0_v, cnt0_v, sem0)

    def pair_body(c2, carry):
        c_even = c2 * 2
        start_fetch(c_even + 1, idx1_v, cnt1_v, sem1)
        wait_fetch(idx0_v, cnt0_v, sem0)
        compute(c_even, idx0_v, cnt0_v)
        start_fetch(c_even + 2, idx0_v, cnt0_v, sem0)
        wait_fetch(idx1_v, cnt1_v, sem1)
        compute(c_even + 1, idx1_v, cnt1_v)
        return carry

    lax.fori_loop(0, NCHUNK // 2, pair_body, 0)
    wait_fetch(idx0_v, cnt0_v, sem0)

    pltpu.sync_copy(res_v, out_hbm.at[pl.ds(wid * ROWS_PER_W, ROWS_PER_W)])


def kernel(domain_indices, counts, embd_weight, weights):
    idx_flat = domain_indices.reshape(B * H)
    cnt_flat = counts.reshape(B * H)
    tab_t = jnp.zeros((E, VPAD), jnp.float32).at[:, :V].set(embd_weight.T)
    tab_flat = tab_t.reshape(E * VPAD)
    w_bcast = jnp.broadcast_to(weights.reshape(E, 1), (E, L)).reshape(E * L)
    out = _dwc_kernel(idx_flat, cnt_flat, tab_flat, w_bcast)
    return out.reshape(B, 1)
